# baseline (speedup 1.0000x reference)
import functools

import jax
import jax.numpy as jnp
from jax import lax
from jax.experimental import pallas as pl
from jax.experimental.pallas import tpu as pltpu
from jax.experimental.pallas import tpu_sc as plsc

_NUM_WORKERS = 32
_TABLE_SIZE = 65536
_LANES = 16
_COLS = 224


@functools.lru_cache(maxsize=None)
def _build_sc_kernel(m: int, rows: int):
    per_w = m // _NUM_WORKERS
    n_chunks = per_w // rows
    n_pairs = n_chunks // 2
    vregs_per_row = _COLS // _LANES
    mesh = plsc.VectorSubcoreMesh(core_axis_name="c", subcore_axis_name="s")

    @functools.partial(
        pl.kernel,
        mesh=mesh,
        out_type=jax.ShapeDtypeStruct((m, _COLS), jnp.float32),
        scratch_types=[
            pltpu.VMEM((_TABLE_SIZE,), jnp.float32),
            pltpu.VMEM((rows, _COLS), jnp.float32),
            pltpu.VMEM((rows, _COLS), jnp.float32),
            pltpu.VMEM((rows, _COLS), jnp.float32),
            pltpu.VMEM((rows, _COLS), jnp.float32),
            pltpu.SemaphoreType.DMA,
            pltpu.SemaphoreType.DMA,
            pltpu.SemaphoreType.DMA,
            pltpu.SemaphoreType.DMA,
        ],
        compiler_params=pltpu.CompilerParams(
            needs_layout_passes=False, use_tc_tiling_on_sc=True),
    )
    def lut_kernel(x_hbm, tab_hbm, out_hbm, tab_v, x0, x1, y0, y1,
                   si0, si1, so0, so1):
        wid = lax.axis_index("s") * 2 + lax.axis_index("c")
        base = wid * per_w
        pltpu.sync_copy(tab_hbm, tab_v)

        def in_copy(j, buf, sem):
            return pltpu.make_async_copy(
                x_hbm.at[pl.ds(base + j * rows, rows), :], buf, sem)

        def out_copy(j, buf, sem):
            return pltpu.make_async_copy(
                buf, out_hbm.at[pl.ds(base + j * rows, rows), :], sem)

        def compute(xb, yb):
            @plsc.parallel_loop(0, rows, step=1, unroll=4)
            def _(r):
                for c in range(vregs_per_row):
                    xv = xb[r, pl.ds(c * _LANES, _LANES)]
                    t = jnp.minimum(jnp.maximum(xv * 4096.0, -32768.0), 32767.0)
                    idx = t.astype(jnp.int32) + 32768
                    yb[r, pl.ds(c * _LANES, _LANES)] = plsc.load_gather(
                        tab_v, [idx])

        in_copy(0, x0, si0).start()

        def body(jj, carry):
            j0 = 2 * jj
            j1 = j0 + 1
            in_copy(j1, x1, si1).start()
            in_copy(j0, x0, si0).wait()

            @pl.when(jj > 0)
            def _():
                out_copy(j0, y0, so0).wait()

            compute(x0, y0)
            out_copy(j0, y0, so0).start()

            @pl.when(jj < n_pairs - 1)
            def _():
                in_copy(j0 + 2, x0, si0).start()

            in_copy(j1, x1, si1).wait()

            @pl.when(jj > 0)
            def _():
                out_copy(j1, y1, so1).wait()

            compute(x1, y1)
            out_copy(j1, y1, so1).start()
            return carry

        lax.fori_loop(0, n_pairs, body, 0)
        out_copy(n_chunks - 2, y0, so0).wait()
        out_copy(n_chunks - 1, y1, so1).wait()

    return lut_kernel


def kernel(x, table):
    tab_q = jnp.clip(jnp.round(table * 128.0), -128.0, 127.0) * (1.0 / 128.0)
    b, ch, h, w = x.shape
    m = b * ch * h
    x2 = x.reshape(m, w)
    out = _build_sc_kernel(m, 64)(x2, tab_q)
    return out.reshape(x.shape)


# speedup vs baseline: 1.1594x; 1.1594x over previous
import functools

import jax
import jax.numpy as jnp
from jax import lax
from jax.experimental import pallas as pl
from jax.experimental.pallas import tpu as pltpu
from jax.experimental.pallas import tpu_sc as plsc

_NUM_WORKERS = 32
_TABLE_SIZE = 65536
_LANES = 16
_COLS = 224


@functools.lru_cache(maxsize=None)
def _build_sc_kernel(m: int, rows: int):
    per_w = m // _NUM_WORKERS
    n_chunks = per_w // rows
    n_pairs = n_chunks // 2
    vregs_per_row = _COLS // _LANES
    mesh = plsc.VectorSubcoreMesh(core_axis_name="c", subcore_axis_name="s")

    @functools.partial(
        pl.kernel,
        mesh=mesh,
        out_type=jax.ShapeDtypeStruct((m, _COLS), jnp.float32),
        scratch_types=[
            pltpu.VMEM((_TABLE_SIZE,), jnp.float32),
            pltpu.VMEM((rows, _COLS), jnp.float32),
            pltpu.VMEM((rows, _COLS), jnp.float32),
            pltpu.VMEM((rows, _COLS), jnp.float32),
            pltpu.VMEM((rows, _COLS), jnp.float32),
            pltpu.SemaphoreType.DMA,
            pltpu.SemaphoreType.DMA,
            pltpu.SemaphoreType.DMA,
            pltpu.SemaphoreType.DMA,
        ],
        compiler_params=pltpu.CompilerParams(
            needs_layout_passes=False, use_tc_tiling_on_sc=True),
    )
    def lut_kernel(x_hbm, tab_hbm, out_hbm, tab_v, x0, x1, y0, y1,
                   si0, si1, so0, so1):
        wid = lax.axis_index("s") * 2 + lax.axis_index("c")
        base = wid * per_w
        pltpu.sync_copy(tab_hbm, tab_v)

        def in_copy(j, buf, sem):
            return pltpu.make_async_copy(
                x_hbm.at[pl.ds(base + j * rows, rows), :], buf, sem)

        def out_copy(j, buf, sem):
            return pltpu.make_async_copy(
                buf, out_hbm.at[pl.ds(base + j * rows, rows), :], sem)

        def compute(xb, yb):
            @plsc.parallel_loop(0, rows, step=1, unroll=1)
            def _(r):
                for c in range(vregs_per_row):
                    xv = xb[r, pl.ds(c * _LANES, _LANES)]
                    t = jnp.minimum(jnp.maximum(xv * 4096.0, -32768.0), 32767.0)
                    idx = t.astype(jnp.int32) + 32768
                    yb[r, pl.ds(c * _LANES, _LANES)] = plsc.load_gather(
                        tab_v, [idx])

        in_copy(0, x0, si0).start()

        def body(jj, carry):
            j0 = 2 * jj
            j1 = j0 + 1
            in_copy(j1, x1, si1).start()
            in_copy(j0, x0, si0).wait()

            @pl.when(jj > 0)
            def _():
                out_copy(j0, y0, so0).wait()

            compute(x0, y0)
            out_copy(j0, y0, so0).start()

            @pl.when(jj < n_pairs - 1)
            def _():
                in_copy(j0 + 2, x0, si0).start()

            in_copy(j1, x1, si1).wait()

            @pl.when(jj > 0)
            def _():
                out_copy(j1, y1, so1).wait()

            compute(x1, y1)
            out_copy(j1, y1, so1).start()
            return carry

        lax.fori_loop(0, n_pairs, body, 0)
        out_copy(n_chunks - 2, y0, so0).wait()
        out_copy(n_chunks - 1, y1, so1).wait()

    return lut_kernel


def kernel(x, table):
    tab_q = jnp.clip(jnp.round(table * 128.0), -128.0, 127.0) * (1.0 / 128.0)
    b, ch, h, w = x.shape
    m = b * ch * h
    x2 = x.reshape(m, w)
    out = _build_sc_kernel(m, 64)(x2, tab_q)
    return out.reshape(x.shape)


# speedup vs baseline: 1.2283x; 1.0594x over previous
import functools

import jax
import jax.numpy as jnp
from jax import lax
from jax.experimental import pallas as pl
from jax.experimental.pallas import tpu as pltpu
from jax.experimental.pallas import tpu_sc as plsc

_NUM_WORKERS = 32
_TABLE_SIZE = 65536
_LANES = 16
_COLS = 224


@functools.lru_cache(maxsize=None)
def _build_sc_kernel(m: int, rows: int):
    per_w = m // _NUM_WORKERS
    n_chunks = per_w // rows
    n_pairs = n_chunks // 2
    vregs_per_row = _COLS // _LANES
    mesh = plsc.VectorSubcoreMesh(core_axis_name="c", subcore_axis_name="s")

    n_quads = n_chunks // 4

    @functools.partial(
        pl.kernel,
        mesh=mesh,
        out_type=jax.ShapeDtypeStruct((m, _COLS), jnp.float32),
        scratch_types=[
            pltpu.VMEM((_TABLE_SIZE,), jnp.float32),
            [pltpu.VMEM((rows, _COLS), jnp.float32)] * 4,
            [pltpu.VMEM((rows, _COLS), jnp.float32)] * 4,
            [pltpu.SemaphoreType.DMA] * 4,
            [pltpu.SemaphoreType.DMA] * 4,
        ],
        compiler_params=pltpu.CompilerParams(
            needs_layout_passes=False, use_tc_tiling_on_sc=True),
    )
    def lut_kernel(x_hbm, tab_hbm, out_hbm, tab_v, xb, yb, si, so):
        wid = lax.axis_index("s") * 2 + lax.axis_index("c")
        base = wid * per_w
        pltpu.sync_copy(tab_hbm, tab_v)

        def in_copy(j, buf, sem):
            return pltpu.make_async_copy(
                x_hbm.at[pl.ds(base + j * rows, rows), :], buf, sem)

        def out_copy(j, buf, sem):
            return pltpu.make_async_copy(
                buf, out_hbm.at[pl.ds(base + j * rows, rows), :], sem)

        def compute(xv_ref, yv_ref):
            @plsc.parallel_loop(0, rows, step=1, unroll=2)
            def _(r):
                for c in range(vregs_per_row):
                    xv = xv_ref[r, pl.ds(c * _LANES, _LANES)]
                    t = jnp.minimum(jnp.maximum(xv * 4096.0, -32768.0), 32767.0)
                    idx = t.astype(jnp.int32) + 32768
                    yv_ref[r, pl.ds(c * _LANES, _LANES)] = plsc.load_gather(
                        tab_v, [idx])

        for b in range(3):
            in_copy(b, xb[b], si[b]).start()

        def body(jj, carry):
            for b in range(4):
                j = 4 * jj + b

                @pl.when(j + 3 < n_chunks)
                def _():
                    in_copy(j + 3, xb[(b + 3) % 4], si[(b + 3) % 4]).start()

                in_copy(j, xb[b], si[b]).wait()

                @pl.when(jj > 0)
                def _():
                    out_copy(j - 4, yb[b], so[b]).wait()

                compute(xb[b], yb[b])
                out_copy(j, yb[b], so[b]).start()
            return carry

        lax.fori_loop(0, n_quads, body, 0)
        for b in range(4):
            out_copy(n_chunks - 4 + b, yb[b], so[b]).wait()

    return lut_kernel


def kernel(x, table):
    tab_q = jnp.clip(jnp.round(table * 128.0), -128.0, 127.0) * (1.0 / 128.0)
    b, ch, h, w = x.shape
    m = b * ch * h
    x2 = x.reshape(m, w)
    out = _build_sc_kernel(m, 32)(x2, tab_q)
    return out.reshape(x.shape)
